# Initial kernel scaffold; baseline (speedup 1.0000x reference)
import jax
import jax.numpy as jnp
from jax.experimental import pallas as pl


def kernel(entity_pairs, train_edges, labels, entity2edges, edge2entities, edge2relation, W0, b0, W1, b1):
    raise NotImplementedError("write your pallas kernel here")


# speedup vs baseline: 3.1451x; 3.1451x over previous
import functools

import jax
import jax.numpy as jnp
from jax import lax
from jax.experimental import pallas as pl
from jax.experimental.pallas import tpu as pltpu
from jax.experimental.pallas import tpu_sc as plsc

R = 256
S = 8
D = 64
DP = 64
L = 16
CH = 128

_GDN = lax.GatherDimensionNumbers(
    offset_dims=(), collapsed_slice_dims=(0,), start_index_map=(0,))


def _take16(v, idx):
  return lax.gather(v, idx[:, None], _GDN, slice_sizes=(1,),
                    mode=lax.GatherScatterMode.PROMISE_IN_BOUNDS)


def _sc_pooled(ep_flat, te, e2e_flat, e2ent_flat, e2r, w0big, B):
  info = plsc.get_sparse_core_info()
  NC, NS = info.num_cores, info.num_subcores
  NW = NC * NS
  BW = B // NW
  G = BW * L
  NR = L * G
  NCH = NR // CH
  mesh = plsc.VectorSubcoreMesh(core_axis_name="c", subcore_axis_name="s")

  @functools.partial(
      pl.kernel,
      mesh=mesh,
      compiler_params=pltpu.CompilerParams(use_tc_tiling_on_sc=False),
      out_type=jax.ShapeDtypeStruct((B, D), jnp.float32),
      scratch_types=[
          pltpu.VMEM((2 * BW,), jnp.int32),
          pltpu.VMEM((BW,), jnp.int32),
          pltpu.VMEM((G,), jnp.int32),
          pltpu.VMEM((G,), jnp.int32),
          pltpu.VMEM((G,), jnp.int32),
          pltpu.VMEM((2 * G,), jnp.int32),
          pltpu.VMEM((2 * G,), jnp.int32),
          pltpu.VMEM((NR,), jnp.int32),
          pltpu.VMEM((NR,), jnp.int32),
          pltpu.VMEM((NR,), jnp.int32),
          pltpu.VMEM((NR,), jnp.int32),
          pltpu.VMEM((NCH, CH), jnp.int32),
          pltpu.VMEM((G,), jnp.float32),
          pltpu.VMEM((CH, DP), jnp.float32),
          pltpu.VMEM((CH, DP), jnp.float32),
          pltpu.VMEM((G // 2, DP), jnp.float32),
          pltpu.VMEM((BW, D), jnp.float32),
          pltpu.VMEM_SHARED((NS * G, DP), jnp.float32),
          pltpu.SemaphoreType.DMA,
          pltpu.SemaphoreType.DMA,
          pltpu.SemaphoreType.DMA,
      ],
  )
  def k(ep_hbm, te_hbm, e2e_hbm, e2ent_hbm, e2r_hbm, w0_hbm, out_hbm,
        pairs_v, te_v, idxA_v, edges1_v, rel1_v, idxB_v, ents_v,
        idxC_v, edges2_v, rel2_v, idx2_v, sidx_v, mask0f_v,
        stg0_v, stg1_v, acc_v, pooled_v, acc_sh,
        sem_a, sem_b, sem_s):
    iota = lax.iota(jnp.int32, L)
    sid = lax.axis_index("s")
    wid = sid * NC + lax.axis_index("c")
    base = wid * BW
    gbase = sid * G

    pltpu.sync_copy(ep_hbm.at[pl.ds(base * 2, 2 * BW)], pairs_v)
    pltpu.sync_copy(te_hbm.at[pl.ds(base, BW)], te_v)

    def expand8(src_v, dst_v, u, _):
      sv = src_v[pl.ds((u >> 3) * L, L)]
      ev = _take16(sv, 2 * (u & 7) + (iota >> 3))
      dst_v[pl.ds(u * L, L)] = ev * S + (iota & 7)
      return 0

    lax.fori_loop(0, G // L, functools.partial(expand8, pairs_v, idxA_v), 0)
    for c in range(G // CH):
      pltpu.async_copy(e2e_hbm.at[idxA_v.at[pl.ds(c * CH, CH)]],
                       edges1_v.at[pl.ds(c * CH, CH)], sem_a)
    for c in range(G // CH):
      pltpu.make_async_copy(e2e_hbm.at[idxA_v.at[pl.ds(c * CH, CH)]],
                            edges1_v.at[pl.ds(c * CH, CH)], sem_a).wait()

    h_rel1 = [pltpu.async_copy(e2r_hbm.at[edges1_v.at[pl.ds(c * CH, CH)]],
                               rel1_v.at[pl.ds(c * CH, CH)], sem_b)
              for c in range(G // CH)]

    def mk_idxB(u, _):
      sv = edges1_v[pl.ds((u >> 1) * L, L)]
      ev = _take16(sv, S * (u & 1) + (iota >> 1))
      idxB_v[pl.ds(u * L, L)] = ev * 2 + (iota & 1)
      return 0
    lax.fori_loop(0, (2 * G) // L, mk_idxB, 0)

    for c in range(2 * G // CH):
      pltpu.async_copy(e2ent_hbm.at[idxB_v.at[pl.ds(c * CH, CH)]],
                       ents_v.at[pl.ds(c * CH, CH)], sem_a)
    for c in range(2 * G // CH):
      pltpu.make_async_copy(e2ent_hbm.at[idxB_v.at[pl.ds(c * CH, CH)]],
                            ents_v.at[pl.ds(c * CH, CH)], sem_a).wait()

    lax.fori_loop(0, NR // L, functools.partial(expand8, ents_v, idxC_v), 0)
    for c in range(NR // CH):
      pltpu.async_copy(e2e_hbm.at[idxC_v.at[pl.ds(c * CH, CH)]],
                       edges2_v.at[pl.ds(c * CH, CH)], sem_a)
    for c in range(NR // CH):
      pltpu.make_async_copy(e2e_hbm.at[idxC_v.at[pl.ds(c * CH, CH)]],
                            edges2_v.at[pl.ds(c * CH, CH)], sem_a).wait()

    for c in range(NR // CH):
      pltpu.async_copy(e2r_hbm.at[edges2_v.at[pl.ds(c * CH, CH)]],
                       rel2_v.at[pl.ds(c * CH, CH)], sem_a)
    for c in range(NR // CH):
      pltpu.make_async_copy(e2r_hbm.at[edges2_v.at[pl.ds(c * CH, CH)]],
                            rel2_v.at[pl.ds(c * CH, CH)], sem_a).wait()
    for h in h_rel1:
      h.wait()

    def mk_mask(b, _):
      tv = te_v[pl.ds((b >> 4) * L, L)]
      teb = _take16(tv, jnp.full((L,), b & 15, jnp.int32))
      e1v = edges1_v[pl.ds(b * L, L)]
      mask0f_v[pl.ds(b * L, L)] = jnp.where(
          e1v != teb, 1.0, 0.0).astype(jnp.float32)
      r1 = rel1_v[pl.ds(b * L, L)]
      rel1_v[pl.ds(b * L, L)] = r1 + (R + 1)
      return 0
    lax.fori_loop(0, BW, mk_mask, 0)

    def mk_idx2(u, _):
      b = u >> 4
      tv = te_v[pl.ds((u >> 8) * L, L)]
      teb = _take16(tv, jnp.full((L,), b & 15, jnp.int32))
      ev = edges2_v[pl.ds(u * L, L)]
      rv = rel2_v[pl.ds(u * L, L)]
      idx2_v[pl.ds(u * L, L)] = jnp.where(ev == teb, R, rv)
      return 0
    lax.fori_loop(0, NR // L, mk_idx2, 0)

    def mk_sidx(cu, _):
      for c in range(CH // L):
        sidx_v[cu, pl.ds(c * L, L)] = jnp.full(
            (L,), gbase + (CH // L) * cu + c, jnp.int32)
      return 0
    lax.fori_loop(0, NCH, mk_sidx, 0)

    for c in range(G // CH):
      h = pltpu.async_copy(w0_hbm.at[rel1_v.at[pl.ds(c * CH, CH)]],
                           stg0_v if c % 2 == 0 else stg1_v, sem_a)
      h.wait()
      pltpu.sync_copy(stg0_v if c % 2 == 0 else stg1_v,
                      acc_sh.at[pl.ds(gbase + c * CH, CH)])

    stgs = [stg0_v, stg1_v]
    def fire_g(cu):
      return pltpu.async_copy(w0_hbm.at[idx2_v.at[pl.ds(cu * CH, CH)]],
                              stgs[cu % 2], sem_b)
    hg = {0: fire_g(0)}
    for cu in range(NCH):
      if cu >= 1:
        pltpu.make_async_copy(stgs[(cu - 1) % 2],
                              acc_sh.at[sidx_v.at[cu - 1]], sem_s).wait()
      if cu + 1 < NCH:
        hg[cu + 1] = fire_g(cu + 1)
      hg.pop(cu).wait()
      pltpu.async_copy(stgs[cu % 2], acc_sh.at[sidx_v.at[cu]], sem_s,
                       add=True)
    pltpu.make_async_copy(stgs[(NCH - 1) % 2],
                          acc_sh.at[sidx_v.at[NCH - 1]], sem_s).wait()

    for pas in range(2):
      pltpu.sync_copy(acc_sh.at[pl.ds(gbase + pas * (G // 2), G // 2)], acc_v)

      def pool_b(lb, _):
        b = lb + pas * (BW // 2)
        mv = mask0f_v[pl.ds(b * L, L)]
        def pool_j(j, pr):
          row = lb * L + j
          m = _take16(mv, jnp.full((L,), j, jnp.int32))
          out = []
          for c in range(D // L):
            h = jnp.maximum(acc_v[row, pl.ds(c * L, L)] * 0.0625, 0.0)
            out.append(pr[c] + m * h)
          return tuple(out)
        z = jnp.zeros((L,), jnp.float32)
        pr = lax.fori_loop(0, L, pool_j, (z, z, z, z))
        for c in range(D // L):
          pooled_v[b, pl.ds(c * L, L)] = pr[c]
        return 0
      lax.fori_loop(0, BW // 2, pool_b, 0)

    pltpu.sync_copy(pooled_v, out_hbm.at[pl.ds(base, BW)])

  return k(ep_flat, te, e2e_flat, e2ent_flat, e2r, w0big)


def _tc_head(pooled, W1, b1):
  B = pooled.shape[0]

  def body(p_ref, w_ref, b_ref, o_ref):
    p = p_ref[...] * 0.0625
    o_ref[...] = jax.nn.sigmoid(
        jnp.dot(p, w_ref[...], preferred_element_type=jnp.float32)
        + b_ref[...])

  return pl.pallas_call(
      body,
      out_shape=jax.ShapeDtypeStruct((B, R), jnp.float32),
  )(pooled, W1, b1.reshape(1, R))


def kernel(entity_pairs, train_edges, labels, entity2edges, edge2entities,
           edge2relation, W0, b0, W1, b1):
  B = labels.shape[0]
  ep_flat = entity_pairs.reshape(-1).astype(jnp.int32)
  te = train_edges.astype(jnp.int32)
  e2e_flat = entity2edges.reshape(-1).astype(jnp.int32)
  e2ent_flat = edge2entities.reshape(-1).astype(jnp.int32)
  e2r = edge2relation.astype(jnp.int32)
  zrow = jnp.zeros((1, D), jnp.float32)
  w0big = jnp.concatenate(
      [W0, zrow, 16.0 * (W0 + b0[None, :]), (16.0 * b0)[None, :]], axis=0)
  if DP != D:
    w0big = jnp.pad(w0big, ((0, 0), (0, DP - D)))
  pooled = _sc_pooled(ep_flat, te, e2e_flat, e2ent_flat, e2r, w0big, B)
  return _tc_head(pooled, W1, b1)


# speedup vs baseline: 3.1535x; 1.0027x over previous
import functools

import jax
import jax.numpy as jnp
from jax import lax
from jax.experimental import pallas as pl
from jax.experimental.pallas import tpu as pltpu
from jax.experimental.pallas import tpu_sc as plsc

R = 256
S = 8
D = 64
DP = 64
L = 16
CH = 128
CHI = 512

_GDN = lax.GatherDimensionNumbers(
    offset_dims=(), collapsed_slice_dims=(0,), start_index_map=(0,))


def _take16(v, idx):
  return lax.gather(v, idx[:, None], _GDN, slice_sizes=(1,),
                    mode=lax.GatherScatterMode.PROMISE_IN_BOUNDS)


def _sc_pooled(ep_flat, te, e2e_flat, e2ent_flat, e2r, w0big, B):
  info = plsc.get_sparse_core_info()
  NC, NS = info.num_cores, info.num_subcores
  NW = NC * NS
  BW = B // NW
  G = BW * L
  NR = L * G
  NCH = NR // CH
  mesh = plsc.VectorSubcoreMesh(core_axis_name="c", subcore_axis_name="s")

  @functools.partial(
      pl.kernel,
      mesh=mesh,
      compiler_params=pltpu.CompilerParams(use_tc_tiling_on_sc=False),
      out_type=jax.ShapeDtypeStruct((B, D), jnp.float32),
      scratch_types=[
          pltpu.VMEM((2 * BW,), jnp.int32),
          pltpu.VMEM((BW,), jnp.int32),
          pltpu.VMEM((G,), jnp.int32),
          pltpu.VMEM((G,), jnp.int32),
          pltpu.VMEM((G,), jnp.int32),
          pltpu.VMEM((2 * G,), jnp.int32),
          pltpu.VMEM((2 * G,), jnp.int32),
          pltpu.VMEM((NR,), jnp.int32),
          pltpu.VMEM((NR,), jnp.int32),
          pltpu.VMEM((NR,), jnp.int32),
          pltpu.VMEM((NR,), jnp.int32),
          pltpu.VMEM((NCH, CH), jnp.int32),
          pltpu.VMEM((G,), jnp.float32),
          pltpu.VMEM((CH, DP), jnp.float32),
          pltpu.VMEM((CH, DP), jnp.float32),
          pltpu.VMEM((G // 2, DP), jnp.float32),
          pltpu.VMEM((BW, D), jnp.float32),
          pltpu.VMEM_SHARED((NS * G, DP), jnp.float32),
          pltpu.SemaphoreType.DMA,
          pltpu.SemaphoreType.DMA,
          pltpu.SemaphoreType.DMA,
      ],
  )
  def k(ep_hbm, te_hbm, e2e_hbm, e2ent_hbm, e2r_hbm, w0_hbm, out_hbm,
        pairs_v, te_v, idxA_v, edges1_v, rel1_v, idxB_v, ents_v,
        idxC_v, edges2_v, rel2_v, idx2_v, sidx_v, mask0f_v,
        stg0_v, stg1_v, acc_v, pooled_v, acc_sh,
        sem_a, sem_b, sem_s):
    iota = lax.iota(jnp.int32, L)
    sid = lax.axis_index("s")
    wid = sid * NC + lax.axis_index("c")
    base = wid * BW
    gbase = sid * G

    pltpu.sync_copy(ep_hbm.at[pl.ds(base * 2, 2 * BW)], pairs_v)
    pltpu.sync_copy(te_hbm.at[pl.ds(base, BW)], te_v)

    def expand8(src_v, dst_v, u, _):
      sv = src_v[pl.ds((u >> 3) * L, L)]
      ev = _take16(sv, 2 * (u & 7) + (iota >> 3))
      dst_v[pl.ds(u * L, L)] = ev * S + (iota & 7)
      return 0

    lax.fori_loop(0, G // L, functools.partial(expand8, pairs_v, idxA_v), 0)
    pltpu.async_copy(e2e_hbm.at[idxA_v], edges1_v, sem_a).wait()

    h_rel1 = [pltpu.async_copy(e2r_hbm.at[edges1_v], rel1_v, sem_b)]

    def mk_idxB(u, _):
      sv = edges1_v[pl.ds((u >> 1) * L, L)]
      ev = _take16(sv, S * (u & 1) + (iota >> 1))
      idxB_v[pl.ds(u * L, L)] = ev * 2 + (iota & 1)
      return 0
    lax.fori_loop(0, (2 * G) // L, mk_idxB, 0)

    for c in range(2 * G // CHI):
      pltpu.async_copy(e2ent_hbm.at[idxB_v.at[pl.ds(c * CHI, CHI)]],
                       ents_v.at[pl.ds(c * CHI, CHI)], sem_a)
    for c in range(2 * G // CHI):
      pltpu.make_async_copy(e2ent_hbm.at[idxB_v.at[pl.ds(c * CHI, CHI)]],
                            ents_v.at[pl.ds(c * CHI, CHI)], sem_a).wait()

    lax.fori_loop(0, NR // L, functools.partial(expand8, ents_v, idxC_v), 0)
    for c in range(NR // CHI):
      pltpu.async_copy(e2e_hbm.at[idxC_v.at[pl.ds(c * CHI, CHI)]],
                       edges2_v.at[pl.ds(c * CHI, CHI)], sem_a)
    for c in range(NR // CHI):
      pltpu.make_async_copy(e2e_hbm.at[idxC_v.at[pl.ds(c * CHI, CHI)]],
                            edges2_v.at[pl.ds(c * CHI, CHI)], sem_a).wait()
      pltpu.async_copy(e2r_hbm.at[edges2_v.at[pl.ds(c * CHI, CHI)]],
                       rel2_v.at[pl.ds(c * CHI, CHI)], sem_b)
    for c in range(NR // CHI):
      pltpu.make_async_copy(e2r_hbm.at[edges2_v.at[pl.ds(c * CHI, CHI)]],
                            rel2_v.at[pl.ds(c * CHI, CHI)], sem_b).wait()
    for h in h_rel1:
      h.wait()

    def mk_mask(b, _):
      tv = te_v[pl.ds((b >> 4) * L, L)]
      teb = _take16(tv, jnp.full((L,), b & 15, jnp.int32))
      e1v = edges1_v[pl.ds(b * L, L)]
      mask0f_v[pl.ds(b * L, L)] = jnp.where(
          e1v != teb, 1.0, 0.0).astype(jnp.float32)
      r1 = rel1_v[pl.ds(b * L, L)]
      rel1_v[pl.ds(b * L, L)] = r1 + (R + 1)
      return 0
    lax.fori_loop(0, BW, mk_mask, 0)

    def mk_idx2(u, _):
      b = u >> 4
      tv = te_v[pl.ds((u >> 8) * L, L)]
      teb = _take16(tv, jnp.full((L,), b & 15, jnp.int32))
      ev = edges2_v[pl.ds(u * L, L)]
      rv = rel2_v[pl.ds(u * L, L)]
      idx2_v[pl.ds(u * L, L)] = jnp.where(ev == teb, R, rv)
      return 0
    lax.fori_loop(0, NR // L, mk_idx2, 0)

    def mk_sidx(cu, _):
      for c in range(CH // L):
        sidx_v[cu, pl.ds(c * L, L)] = jnp.full(
            (L,), gbase + (CH // L) * cu + c, jnp.int32)
      return 0
    lax.fori_loop(0, NCH, mk_sidx, 0)

    for c in range(G // CH):
      h = pltpu.async_copy(w0_hbm.at[rel1_v.at[pl.ds(c * CH, CH)]],
                           stg0_v if c % 2 == 0 else stg1_v, sem_a)
      h.wait()
      pltpu.sync_copy(stg0_v if c % 2 == 0 else stg1_v,
                      acc_sh.at[pl.ds(gbase + c * CH, CH)])

    stgs = [stg0_v, stg1_v]
    def fire_g(cu):
      return pltpu.async_copy(w0_hbm.at[idx2_v.at[pl.ds(cu * CH, CH)]],
                              stgs[cu % 2], sem_b)
    hg = {0: fire_g(0)}
    for cu in range(NCH):
      if cu >= 1:
        pltpu.make_async_copy(stgs[(cu - 1) % 2],
                              acc_sh.at[sidx_v.at[cu - 1]], sem_s).wait()
      if cu + 1 < NCH:
        hg[cu + 1] = fire_g(cu + 1)
      hg.pop(cu).wait()
      pltpu.async_copy(stgs[cu % 2], acc_sh.at[sidx_v.at[cu]], sem_s,
                       add=True)
    pltpu.make_async_copy(stgs[(NCH - 1) % 2],
                          acc_sh.at[sidx_v.at[NCH - 1]], sem_s).wait()

    for pas in range(2):
      pltpu.sync_copy(acc_sh.at[pl.ds(gbase + pas * (G // 2), G // 2)], acc_v)

      def pool_b(lb, _):
        b = lb + pas * (BW // 2)
        mv = mask0f_v[pl.ds(b * L, L)]
        def pool_j(j, pr):
          row = lb * L + j
          m = _take16(mv, jnp.full((L,), j, jnp.int32))
          out = []
          for c in range(D // L):
            h = jnp.maximum(acc_v[row, pl.ds(c * L, L)] * 0.0625, 0.0)
            out.append(pr[c] + m * h)
          return tuple(out)
        z = jnp.zeros((L,), jnp.float32)
        pr = lax.fori_loop(0, L, pool_j, (z, z, z, z))
        for c in range(D // L):
          pooled_v[b, pl.ds(c * L, L)] = pr[c]
        return 0
      lax.fori_loop(0, BW // 2, pool_b, 0)

    pltpu.sync_copy(pooled_v, out_hbm.at[pl.ds(base, BW)])

  return k(ep_flat, te, e2e_flat, e2ent_flat, e2r, w0big)


def _tc_head(pooled, W1, b1):
  B = pooled.shape[0]

  def body(p_ref, w_ref, b_ref, o_ref):
    p = p_ref[...] * 0.0625
    o_ref[...] = jax.nn.sigmoid(
        jnp.dot(p, w_ref[...], preferred_element_type=jnp.float32)
        + b_ref[...])

  return pl.pallas_call(
      body,
      out_shape=jax.ShapeDtypeStruct((B, R), jnp.float32),
  )(pooled, W1, b1.reshape(1, R))


def kernel(entity_pairs, train_edges, labels, entity2edges, edge2entities,
           edge2relation, W0, b0, W1, b1):
  B = labels.shape[0]
  ep_flat = entity_pairs.reshape(-1).astype(jnp.int32)
  te = train_edges.astype(jnp.int32)
  e2e_flat = entity2edges.reshape(-1).astype(jnp.int32)
  e2ent_flat = edge2entities.reshape(-1).astype(jnp.int32)
  e2r = edge2relation.astype(jnp.int32)
  zrow = jnp.zeros((1, D), jnp.float32)
  w0big = jnp.concatenate(
      [W0, zrow, 16.0 * (W0 + b0[None, :]), (16.0 * b0)[None, :]], axis=0)
  if DP != D:
    w0big = jnp.pad(w0big, ((0, 0), (0, DP - D)))
  pooled = _sc_pooled(ep_flat, te, e2e_flat, e2ent_flat, e2r, w0big, B)
  return _tc_head(pooled, W1, b1)


# speedup vs baseline: 7.6223x; 2.4171x over previous
import functools

import jax
import jax.numpy as jnp
from jax import lax
from jax.experimental import pallas as pl
from jax.experimental.pallas import tpu as pltpu
from jax.experimental.pallas import tpu_sc as plsc

R = 256
S = 8
D = 64
DP = 64
L = 16
CH = 128
CHI = 512

_GDN = lax.GatherDimensionNumbers(
    offset_dims=(), collapsed_slice_dims=(0,), start_index_map=(0,))


def _take16(v, idx):
  return lax.gather(v, idx[:, None], _GDN, slice_sizes=(1,),
                    mode=lax.GatherScatterMode.PROMISE_IN_BOUNDS)


def _sc_pooled(ep_flat, te, e2e_flat, e2ent_flat, e2r, w0big, B):
  info = plsc.get_sparse_core_info()
  NC, NS = info.num_cores, info.num_subcores
  NW = NC * NS
  BW = B // NW
  G = BW * L
  NR = L * G
  NCH = NR // CH
  mesh = plsc.VectorSubcoreMesh(core_axis_name="c", subcore_axis_name="s")

  @functools.partial(
      pl.kernel,
      mesh=mesh,
      compiler_params=pltpu.CompilerParams(use_tc_tiling_on_sc=False),
      out_type=jax.ShapeDtypeStruct((B, D), jnp.float32),
      scratch_types=[
          pltpu.VMEM((BW,), jnp.int32),
          pltpu.VMEM((2 * BW,), jnp.int32),
          pltpu.VMEM((BW,), jnp.int32),
          pltpu.VMEM((2 * BW, L), jnp.int32),
          pltpu.VMEM((G,), jnp.int32),
          pltpu.VMEM((G,), jnp.int32),
          pltpu.VMEM((G,), jnp.int32),
          pltpu.VMEM((2 * G,), jnp.int32),
          pltpu.VMEM((2 * G, L), jnp.int32),
          pltpu.VMEM((NR,), jnp.int32),
          pltpu.VMEM((NR,), jnp.int32),
          pltpu.VMEM((NR,), jnp.int32),
          pltpu.VMEM((NCH, CH), jnp.int32),
          pltpu.VMEM((G,), jnp.float32),
          pltpu.VMEM((CH, DP), jnp.float32),
          pltpu.VMEM((CH, DP), jnp.float32),
          pltpu.VMEM((G // 2, DP), jnp.float32),
          pltpu.VMEM((BW, D), jnp.float32),
          pltpu.VMEM_SHARED((NS * G, DP), jnp.float32),
          pltpu.SemaphoreType.DMA,
          pltpu.SemaphoreType.DMA,
          pltpu.SemaphoreType.DMA,
      ],
  )
  def k(ep_hbm, te_hbm, e2e_hbm, e2ent_hbm, e2r_hbm, w0_hbm, out_hbm,
        eppk_v, pairsf_v, te_v, edges1_d, edges1f_v, rel1_v, entspk_v,
        entsf_v, edges2_d, edges2f_v, rel2_v, idx2_v, sidx_v, mask0f_v,
        stg0_v, stg1_v, acc_v, pooled_v, acc_sh,
        sem_a, sem_b, sem_s):
    iota = lax.iota(jnp.int32, L)
    sid = lax.axis_index("s")
    wid = sid * NC + lax.axis_index("c")
    base = wid * BW
    gbase = sid * G

    def unpack_interleave(src1d, dst1d, nout):
      def up(u, _):
        pv = src1d[pl.ds((u >> 1) * L, L)]
        sv = _take16(pv, S * (u & 1) + (iota >> 1))
        hi = lax.shift_right_logical(sv, 16)
        dst1d[pl.ds(u * L, L)] = jnp.where((iota & 1) == 1, hi, sv & 0xffff)
        return 0
      lax.fori_loop(0, nout // L, up, 0)

    def dup_rows_flatten(src2d, dst1d, nrows, roff=0):
      def cp(u, _):
        r0 = src2d[roff + 2 * u, :]
        r1 = src2d[roff + 2 * u + 1, :]
        dst1d[pl.ds(roff * S + u * L, L)] = jnp.where(iota < S, r0, r1)
        return 0
      lax.fori_loop(0, nrows // 2, cp, 0)

    pltpu.sync_copy(ep_hbm.at[pl.ds(base, BW)], eppk_v)
    pltpu.sync_copy(te_hbm.at[pl.ds(base, BW)], te_v)
    unpack_interleave(eppk_v, pairsf_v, 2 * BW)

    pltpu.async_copy(e2e_hbm.at[pairsf_v], edges1_d, sem_a).wait()
    dup_rows_flatten(edges1_d, edges1f_v, 2 * BW)

    h_rel1 = [pltpu.async_copy(e2r_hbm.at[edges1f_v], rel1_v, sem_b)]

    pltpu.async_copy(e2ent_hbm.at[edges1f_v], entspk_v, sem_a).wait()
    unpack_interleave(entspk_v, entsf_v, 2 * G)

    for c in range(2 * G // CHI):
      pltpu.async_copy(
          e2e_hbm.at[entsf_v.at[pl.ds(c * CHI, CHI)]],
          edges2_d.at[pl.ds(c * CHI, CHI)], sem_a)
    for c in range(2 * G // CHI):
      pltpu.make_async_copy(
          e2e_hbm.at[entsf_v.at[pl.ds(c * CHI, CHI)]],
          edges2_d.at[pl.ds(c * CHI, CHI)], sem_a).wait()
      dup_rows_flatten(edges2_d, edges2f_v, CHI, roff=c * CHI)
      for cc in range(CHI * S // 512):
        o = c * CHI * S + cc * 512
        pltpu.async_copy(e2r_hbm.at[edges2f_v.at[pl.ds(o, 512)]],
                         rel2_v.at[pl.ds(o, 512)], sem_b)
    for c in range(NR // 512):
      pltpu.make_async_copy(e2r_hbm.at[edges2f_v.at[pl.ds(c * 512, 512)]],
                            rel2_v.at[pl.ds(c * 512, 512)], sem_b).wait()
    for h in h_rel1:
      h.wait()

    def mk_mask(b, _):
      tv = te_v[pl.ds((b >> 4) * L, L)]
      teb = _take16(tv, jnp.full((L,), b & 15, jnp.int32))
      e1v = edges1f_v[pl.ds(b * L, L)]
      mask0f_v[pl.ds(b * L, L)] = jnp.where(
          e1v != teb, 1.0, 0.0).astype(jnp.float32)
      r1 = rel1_v[pl.ds(b * L, L)]
      rel1_v[pl.ds(b * L, L)] = r1 + (R + 1)
      return 0
    lax.fori_loop(0, BW, mk_mask, 0)

    def mk_idx2(u, _):
      b = u >> 4
      tv = te_v[pl.ds((u >> 8) * L, L)]
      teb = _take16(tv, jnp.full((L,), b & 15, jnp.int32))
      ev = edges2f_v[pl.ds(u * L, L)]
      rv = rel2_v[pl.ds(u * L, L)]
      idx2_v[pl.ds(u * L, L)] = jnp.where(ev == teb, R, rv)
      return 0
    lax.fori_loop(0, NR // L, mk_idx2, 0)

    def mk_sidx(cu, _):
      for c in range(CH // L):
        sidx_v[cu, pl.ds(c * L, L)] = jnp.full(
            (L,), gbase + (CH // L) * cu + c, jnp.int32)
      return 0
    lax.fori_loop(0, NCH, mk_sidx, 0)

    for c in range(G // CH):
      h = pltpu.async_copy(w0_hbm.at[rel1_v.at[pl.ds(c * CH, CH)]],
                           stg0_v if c % 2 == 0 else stg1_v, sem_a)
      h.wait()
      pltpu.sync_copy(stg0_v if c % 2 == 0 else stg1_v,
                      acc_sh.at[pl.ds(gbase + c * CH, CH)])

    stgs = [stg0_v, stg1_v]
    def fire_g(cu):
      return pltpu.async_copy(w0_hbm.at[idx2_v.at[pl.ds(cu * CH, CH)]],
                              stgs[cu % 2], sem_b)
    hg = {0: fire_g(0)}
    for cu in range(NCH):
      if cu >= 1:
        pltpu.make_async_copy(stgs[(cu - 1) % 2],
                              acc_sh.at[sidx_v.at[cu - 1]], sem_s).wait()
      if cu + 1 < NCH:
        hg[cu + 1] = fire_g(cu + 1)
      hg.pop(cu).wait()
      pltpu.async_copy(stgs[cu % 2], acc_sh.at[sidx_v.at[cu]], sem_s,
                       add=True)
    pltpu.make_async_copy(stgs[(NCH - 1) % 2],
                          acc_sh.at[sidx_v.at[NCH - 1]], sem_s).wait()

    for pas in range(2):
      pltpu.sync_copy(acc_sh.at[pl.ds(gbase + pas * (G // 2), G // 2)], acc_v)

      def pool_b(lb, _):
        b = lb + pas * (BW // 2)
        mv = mask0f_v[pl.ds(b * L, L)]
        def pool_j(j, pr):
          row = lb * L + j
          m = _take16(mv, jnp.full((L,), j, jnp.int32))
          out = []
          for c in range(D // L):
            h = jnp.maximum(acc_v[row, pl.ds(c * L, L)] * 0.0625, 0.0)
            out.append(pr[c] + m * h)
          return tuple(out)
        z = jnp.zeros((L,), jnp.float32)
        pr = lax.fori_loop(0, L, pool_j, (z, z, z, z))
        for c in range(D // L):
          pooled_v[b, pl.ds(c * L, L)] = pr[c]
        return 0
      lax.fori_loop(0, BW // 2, pool_b, 0)

    pltpu.sync_copy(pooled_v, out_hbm.at[pl.ds(base, BW)])

  return k(ep_flat, te, e2e_flat, e2ent_flat, e2r, w0big)


def _tc_head(pooled, W1, b1):
  B = pooled.shape[0]

  def body(p_ref, w_ref, b_ref, o_ref):
    p = p_ref[...] * 0.0625
    o_ref[...] = jax.nn.sigmoid(
        jnp.dot(p, w_ref[...], preferred_element_type=jnp.float32)
        + b_ref[...])

  return pl.pallas_call(
      body,
      out_shape=jax.ShapeDtypeStruct((B, R), jnp.float32),
  )(pooled, W1, b1.reshape(1, R))


def kernel(entity_pairs, train_edges, labels, entity2edges, edge2entities,
           edge2relation, W0, b0, W1, b1):
  B = labels.shape[0]
  te = train_edges.astype(jnp.int32)
  e2r = edge2relation.astype(jnp.int32)
  ep = entity_pairs.astype(jnp.int32)
  e2e = entity2edges.astype(jnp.int32)
  e2ent = edge2entities.astype(jnp.int32)
  ep_pk = ep[:, 0] | (ep[:, 1] << 16)
  e2ent_pk = e2ent[:, 0] | (e2ent[:, 1] << 16)
  e2e16 = jnp.concatenate([e2e, e2e], axis=1)
  zrow = jnp.zeros((1, D), jnp.float32)
  w0big = jnp.concatenate(
      [W0, zrow, 16.0 * (W0 + b0[None, :]), (16.0 * b0)[None, :]], axis=0)
  if DP != D:
    w0big = jnp.pad(w0big, ((0, 0), (0, DP - D)))
  pooled = _sc_pooled(ep_pk, te, e2e16, e2ent_pk, e2r, w0big, B)
  return _tc_head(pooled, W1, b1)
